# baseline (speedup 1.0000x reference)
import functools

import jax
import jax.numpy as jnp
from jax.experimental import pallas as pl
from jax.experimental.pallas import tpu as pltpu

_DELTA = 0.2
_INT_MIN = -2147483648

_ROWS = 16
_AW = 1024


def _body(t_ref, x_ref, out_ref, s_ref, *, n, num_k, inv_m):
    i = pl.program_id(0)
    tgt = t_ref[...]
    int_min = jnp.int32(_INT_MIN)
    nch = n // _AW
    tail0 = nch * _AW
    tw = n - tail0

    def to_sortable(xm):
        bits = jax.lax.bitcast_convert_type(xm, jnp.int32)
        return jnp.where(bits >= 0, bits, jnp.bitwise_not(bits) ^ int_min)

    def from_sortable(s):
        bits = jnp.where(s >= 0, s, jnp.bitwise_not(s ^ int_min))
        return jax.lax.bitcast_convert_type(bits, jnp.float32)

    def vsum(v, w):
        parts = [v[:, j * 128:(j + 1) * 128] for j in range(w // 128)]
        while len(parts) > 1:
            nxt = [a + b for a, b in zip(parts[::2], parts[1::2])]
            if len(parts) % 2:
                nxt.append(parts[-1])
            parts = nxt
        return parts[0]

    def lsum(v):
        return jnp.sum(v, axis=1, keepdims=True)

    pos_acc = jnp.zeros((_ROWS, 128), jnp.float32)
    for c in range(nch):
        x = x_ref[:, c * _AW:(c + 1) * _AW]
        col = jax.lax.broadcasted_iota(jnp.int32, (_ROWS, _AW), 1) + c * _AW
        is_t = col == tgt
        xm = jnp.where(is_t, jnp.float32(-1e30), x)
        s_ref[:, c * _AW:(c + 1) * _AW] = to_sortable(xm)
        pos_acc = pos_acc + vsum(jnp.where(is_t, x, 0.0), _AW)
    xt = x_ref[:, tail0:n]
    colt = jax.lax.broadcasted_iota(jnp.int32, (_ROWS, tw), 1) + tail0
    is_tt = colt == tgt
    xmt = jnp.where(is_tt, jnp.float32(-1e30), xt)
    s_ref[:, tail0:n] = to_sortable(xmt)
    pos = lsum(pos_acc) + lsum(jnp.where(is_tt, xt, 0.0))

    def count_ge(thr):
        acc = jnp.zeros((_ROWS, _AW), jnp.int32)
        for c in range(nch):
            blk = s_ref[:, c * _AW:(c + 1) * _AW]
            acc = jnp.where(blk >= thr, acc + 1, acc)
        st = s_ref[:, tail0:n]
        tcnt = lsum(jnp.where(st >= thr, jnp.int32(1), jnp.int32(0)))
        return lsum(vsum(acc, _AW)) + tcnt

    def rcond(carry):
        b, _, done = carry
        return jnp.logical_and(b < 32, jnp.min(done) == 0)

    def rstep(carry):
        b, prefix, done = carry
        bit = jnp.left_shift(jnp.int32(1), 31 - b)
        cand = prefix | bit
        cnt = count_ge(cand ^ int_min)
        live_take = jnp.logical_and(cnt >= num_k, done == 0)
        new_prefix = jnp.where(live_take, cand, prefix)
        new_done = jnp.where(cnt == num_k, jnp.int32(1), done)
        return b + 1, new_prefix, new_done

    _, prefix, _ = jax.lax.while_loop(
        rcond, rstep,
        (jnp.int32(0), jnp.zeros((_ROWS, 1), jnp.int32),
         jnp.zeros((_ROWS, 1), jnp.int32)))
    thr = prefix ^ int_min

    one = jnp.int32(1)
    zero = jnp.int32(0)
    cgt = jnp.zeros((_ROWS, 128), jnp.int32)
    ceq = jnp.zeros((_ROWS, 128), jnp.int32)
    sgt = jnp.zeros((_ROWS, 128), jnp.float32)
    seq = jnp.zeros((_ROWS, 128), jnp.float32)
    for c in range(nch):
        sblk = s_ref[:, c * _AW:(c + 1) * _AW]
        f = (1.0 + from_sortable(sblk)) ** 2
        gt = sblk > thr
        eq = sblk == thr
        cgt = cgt + vsum(jnp.where(gt, one, zero), _AW)
        ceq = ceq + vsum(jnp.where(eq, one, zero), _AW)
        sgt = sgt + vsum(jnp.where(gt, f, 0.0), _AW)
        seq = seq + vsum(jnp.where(eq, f, 0.0), _AW)
    st = s_ref[:, tail0:n]
    ft = (1.0 + xmt) ** 2
    gtt = st > thr
    eqt = st == thr
    cnt_gt = lsum(cgt) + lsum(jnp.where(gtt, one, zero))
    cnt_eq = lsum(ceq) + lsum(jnp.where(eqt, one, zero))
    sum_gt = lsum(sgt) + lsum(jnp.where(gtt, ft, 0.0))
    sum_eq = lsum(seq) + lsum(jnp.where(eqt, ft, 0.0))

    need = (num_k - cnt_gt).astype(jnp.float32)
    safe_eq = jnp.maximum(cnt_eq, 1).astype(jnp.float32)
    top_sum = sum_gt + jnp.where(need > 0, sum_eq * need / safe_eq, 0.0)
    per_row = (1.0 - pos) ** 2 + (_DELTA / num_k) * top_sum
    blk = jnp.sum(per_row) * inv_m

    @pl.when(i == 0)
    def _():
        out_ref[...] = jnp.zeros_like(out_ref)

    out_ref[...] += blk


def kernel(inputs, targets):
    m, n = inputs.shape
    num_k = int(0.01 * (n - 1))
    t2 = targets.astype(jnp.int32).reshape(m, 1)
    body = functools.partial(_body, n=n, num_k=num_k, inv_m=1.0 / m)
    out = pl.pallas_call(
        body,
        grid=(m // _ROWS,),
        in_specs=[
            pl.BlockSpec((_ROWS, 1), lambda i: (i, 0)),
            pl.BlockSpec((_ROWS, n), lambda i: (i, 0)),
        ],
        out_specs=pl.BlockSpec((1, 1), lambda i: (0, 0)),
        out_shape=jax.ShapeDtypeStruct((1, 1), jnp.float32),
        scratch_shapes=[pltpu.VMEM((_ROWS, n), jnp.int32)],
    )(t2, inputs)
    return out[0, 0]


# speedup vs baseline: 1.1723x; 1.1723x over previous
import functools

import jax
import jax.numpy as jnp
from jax.experimental import pallas as pl
from jax.experimental.pallas import tpu as pltpu

_DELTA = 0.2
_INT_MIN = -2147483648

_ROWS = 16
_AW = 1024


def _body(t_ref, x_ref, out_ref, s_ref, *, n, num_k, inv_m):
    i = pl.program_id(0)
    tgt = t_ref[...]
    int_min = jnp.int32(_INT_MIN)
    nch = n // _AW
    tail0 = nch * _AW
    tw = n - tail0

    def to_sortable(xm):
        bits = jax.lax.bitcast_convert_type(xm, jnp.int32)
        return jnp.where(bits >= 0, bits, jnp.bitwise_not(bits) ^ int_min)

    def from_sortable(s):
        bits = jnp.where(s >= 0, s, jnp.bitwise_not(s ^ int_min))
        return jax.lax.bitcast_convert_type(bits, jnp.float32)

    def vred(v, w, op):
        parts = [v[:, j * 128:(j + 1) * 128] for j in range(w // 128)]
        while len(parts) > 1:
            nxt = [op(a, b) for a, b in zip(parts[::2], parts[1::2])]
            if len(parts) % 2:
                nxt.append(parts[-1])
            parts = nxt
        return parts[0]

    def lsum(v):
        return jnp.sum(v, axis=1, keepdims=True)

    iota0 = jax.lax.broadcasted_iota(jnp.int32, (_ROWS, _AW), 1)
    pos_w = jnp.zeros((_ROWS, _AW), jnp.float32)
    mx_w = jnp.full((_ROWS, _AW), _INT_MIN, jnp.int32)
    for c in range(nch):
        x = x_ref[:, c * _AW:(c + 1) * _AW]
        is_t = iota0 == tgt - c * _AW
        xm = jnp.where(is_t, jnp.float32(-1e30), x)
        s = to_sortable(xm)
        s_ref[:, c * _AW:(c + 1) * _AW] = s
        pos_w = jnp.where(is_t, pos_w + x, pos_w)
        mx_w = jnp.maximum(mx_w, s)
    xt = x_ref[:, tail0:n]
    is_tt = jax.lax.broadcasted_iota(jnp.int32, (_ROWS, tw), 1) == tgt - tail0
    xmt = jnp.where(is_tt, jnp.float32(-1e30), xt)
    st_pro = to_sortable(xmt)
    s_ref[:, tail0:n] = st_pro
    pos = (lsum(vred(pos_w, _AW, lambda a, b: a + b))
           + lsum(jnp.where(is_tt, xt, 0.0)))
    mx = jnp.maximum(
        jnp.max(vred(mx_w, _AW, jnp.maximum), axis=1, keepdims=True),
        jnp.max(st_pro, axis=1, keepdims=True))

    def count_ge(thr):
        acc = jnp.zeros((_ROWS, _AW), jnp.int32)
        for c in range(nch):
            blk = s_ref[:, c * _AW:(c + 1) * _AW]
            acc = jnp.where(blk >= thr, acc + 1, acc)
        st = s_ref[:, tail0:n]
        tcnt = lsum(jnp.where(st >= thr, jnp.int32(1), jnp.int32(0)))
        return lsum(vred(acc, _AW, lambda a, b: a + b)) + tcnt

    def rcond(carry):
        b, _, done = carry
        return jnp.logical_and(b < 32, jnp.min(done) == 0)

    def rstep(carry):
        b, prefix, done = carry
        bit = jnp.left_shift(jnp.int32(1), 31 - b)
        cand = prefix | bit
        thr_c = cand ^ int_min
        run = jnp.any(jnp.logical_and(thr_c <= mx, done == 0))
        cnt = jax.lax.cond(
            run, lambda: count_ge(thr_c),
            lambda: jnp.zeros((_ROWS, 1), jnp.int32))
        live_take = jnp.logical_and(cnt >= num_k, done == 0)
        new_prefix = jnp.where(live_take, cand, prefix)
        new_done = jnp.where(cnt == num_k, jnp.int32(1), done)
        return b + 1, new_prefix, new_done

    _, prefix, _ = jax.lax.while_loop(
        rcond, rstep,
        (jnp.int32(0), jnp.zeros((_ROWS, 1), jnp.int32),
         jnp.zeros((_ROWS, 1), jnp.int32)))
    thr = prefix ^ int_min

    cgt_w = jnp.zeros((_ROWS, _AW), jnp.int32)
    sgt_w = jnp.zeros((_ROWS, _AW), jnp.float32)
    for c in range(nch):
        sblk = s_ref[:, c * _AW:(c + 1) * _AW]
        x = x_ref[:, c * _AW:(c + 1) * _AW]
        f = (1.0 + x) ** 2
        gt = sblk > thr
        cgt_w = jnp.where(gt, cgt_w + 1, cgt_w)
        sgt_w = jnp.where(gt, sgt_w + f, sgt_w)
    st = s_ref[:, tail0:n]
    ft = (1.0 + xt) ** 2
    gtt = st > thr
    cnt_gt = (lsum(vred(cgt_w, _AW, lambda a, b: a + b))
              + lsum(jnp.where(gtt, jnp.int32(1), jnp.int32(0))))
    sum_gt = (lsum(vred(sgt_w, _AW, lambda a, b: a + b))
              + lsum(jnp.where(gtt, ft, 0.0)))

    t_f = from_sortable(thr)
    need = (num_k - cnt_gt).astype(jnp.float32)
    top_sum = sum_gt + need * (1.0 + t_f) ** 2
    per_row = (1.0 - pos) ** 2 + (_DELTA / num_k) * top_sum
    blk = jnp.sum(per_row) * inv_m

    @pl.when(i == 0)
    def _():
        out_ref[...] = jnp.zeros_like(out_ref)

    out_ref[...] += blk


def kernel(inputs, targets):
    m, n = inputs.shape
    num_k = int(0.01 * (n - 1))
    t2 = targets.astype(jnp.int32).reshape(m, 1)
    body = functools.partial(_body, n=n, num_k=num_k, inv_m=1.0 / m)
    out = pl.pallas_call(
        body,
        grid=(m // _ROWS,),
        in_specs=[
            pl.BlockSpec((_ROWS, 1), lambda i: (i, 0)),
            pl.BlockSpec((_ROWS, n), lambda i: (i, 0)),
        ],
        out_specs=pl.BlockSpec((1, 1), lambda i: (0, 0)),
        out_shape=jax.ShapeDtypeStruct((1, 1), jnp.float32),
        scratch_shapes=[pltpu.VMEM((_ROWS, n), jnp.int32)],
    )(t2, inputs)
    return out[0, 0]


# speedup vs baseline: 1.2239x; 1.0440x over previous
import functools

import jax
import jax.numpy as jnp
from jax.experimental import pallas as pl
from jax.experimental.pallas import tpu as pltpu

_DELTA = 0.2
_INT_MIN = -2147483648

_ROWS = 32
_AW = 1024


def _body(t_ref, x_ref, out_ref, s_ref, *, n, num_k, inv_m):
    i = pl.program_id(0)
    tgt = t_ref[...]
    int_min = jnp.int32(_INT_MIN)
    nch = n // _AW
    tail0 = nch * _AW
    tw = n - tail0

    def to_sortable(xm):
        bits = jax.lax.bitcast_convert_type(xm, jnp.int32)
        return jnp.where(bits >= 0, bits, jnp.bitwise_not(bits) ^ int_min)

    def from_sortable(s):
        bits = jnp.where(s >= 0, s, jnp.bitwise_not(s ^ int_min))
        return jax.lax.bitcast_convert_type(bits, jnp.float32)

    def vred(v, w, op):
        parts = [v[:, j * 128:(j + 1) * 128] for j in range(w // 128)]
        while len(parts) > 1:
            nxt = [op(a, b) for a, b in zip(parts[::2], parts[1::2])]
            if len(parts) % 2:
                nxt.append(parts[-1])
            parts = nxt
        return parts[0]

    def lsum(v):
        return jnp.sum(v, axis=1, keepdims=True)

    iota0 = jax.lax.broadcasted_iota(jnp.int32, (_ROWS, _AW), 1)
    pos_w = jnp.zeros((_ROWS, _AW), jnp.float32)
    mx_w = jnp.full((_ROWS, _AW), _INT_MIN, jnp.int32)
    for c in range(nch):
        x = x_ref[:, c * _AW:(c + 1) * _AW]
        is_t = iota0 == tgt - c * _AW
        xm = jnp.where(is_t, jnp.float32(-1e30), x)
        s = to_sortable(xm)
        s_ref[:, c * _AW:(c + 1) * _AW] = s
        pos_w = jnp.where(is_t, pos_w + x, pos_w)
        mx_w = jnp.maximum(mx_w, s)
    xt = x_ref[:, tail0:n]
    is_tt = jax.lax.broadcasted_iota(jnp.int32, (_ROWS, tw), 1) == tgt - tail0
    xmt = jnp.where(is_tt, jnp.float32(-1e30), xt)
    st_pro = to_sortable(xmt)
    s_ref[:, tail0:n] = st_pro
    pos = (lsum(vred(pos_w, _AW, lambda a, b: a + b))
           + lsum(jnp.where(is_tt, xt, 0.0)))
    mx = jnp.maximum(
        jnp.max(vred(mx_w, _AW, jnp.maximum), axis=1, keepdims=True),
        jnp.max(st_pro, axis=1, keepdims=True))

    def count_ge(thr):
        acc = jnp.zeros((_ROWS, _AW), jnp.int32)
        for c in range(nch):
            blk = s_ref[:, c * _AW:(c + 1) * _AW]
            acc = jnp.where(blk >= thr, acc + 1, acc)
        st = s_ref[:, tail0:n]
        tcnt = lsum(jnp.where(st >= thr, jnp.int32(1), jnp.int32(0)))
        return lsum(vred(acc, _AW, lambda a, b: a + b)) + tcnt

    def rcond(carry):
        b, _, done = carry
        return jnp.logical_and(b < 32, jnp.min(done) == 0)

    def rstep(carry):
        b, prefix, done = carry
        bit = jnp.left_shift(jnp.int32(1), 31 - b)
        cand = prefix | bit
        thr_c = cand ^ int_min
        run = jnp.any(jnp.logical_and(thr_c <= mx, done == 0))
        cnt = jax.lax.cond(
            run, lambda: count_ge(thr_c),
            lambda: jnp.zeros((_ROWS, 1), jnp.int32))
        live_take = jnp.logical_and(cnt >= num_k, done == 0)
        new_prefix = jnp.where(live_take, cand, prefix)
        new_done = jnp.where(cnt == num_k, jnp.int32(1), done)
        return b + 1, new_prefix, new_done

    _, prefix, _ = jax.lax.while_loop(
        rcond, rstep,
        (jnp.int32(0), jnp.zeros((_ROWS, 1), jnp.int32),
         jnp.zeros((_ROWS, 1), jnp.int32)))
    thr = prefix ^ int_min

    cgt_w = jnp.zeros((_ROWS, _AW), jnp.int32)
    sgt_w = jnp.zeros((_ROWS, _AW), jnp.float32)
    for c in range(nch):
        sblk = s_ref[:, c * _AW:(c + 1) * _AW]
        x = x_ref[:, c * _AW:(c + 1) * _AW]
        f = (1.0 + x) ** 2
        gt = sblk > thr
        cgt_w = jnp.where(gt, cgt_w + 1, cgt_w)
        sgt_w = jnp.where(gt, sgt_w + f, sgt_w)
    st = s_ref[:, tail0:n]
    ft = (1.0 + xt) ** 2
    gtt = st > thr
    cnt_gt = (lsum(vred(cgt_w, _AW, lambda a, b: a + b))
              + lsum(jnp.where(gtt, jnp.int32(1), jnp.int32(0))))
    sum_gt = (lsum(vred(sgt_w, _AW, lambda a, b: a + b))
              + lsum(jnp.where(gtt, ft, 0.0)))

    t_f = from_sortable(thr)
    need = (num_k - cnt_gt).astype(jnp.float32)
    top_sum = sum_gt + need * (1.0 + t_f) ** 2
    per_row = (1.0 - pos) ** 2 + (_DELTA / num_k) * top_sum
    blk = jnp.sum(per_row) * inv_m

    @pl.when(i == 0)
    def _():
        out_ref[...] = jnp.zeros_like(out_ref)

    out_ref[...] += blk


def kernel(inputs, targets):
    m, n = inputs.shape
    num_k = int(0.01 * (n - 1))
    t2 = targets.astype(jnp.int32).reshape(m, 1)
    body = functools.partial(_body, n=n, num_k=num_k, inv_m=1.0 / m)
    out = pl.pallas_call(
        body,
        grid=(m // _ROWS,),
        in_specs=[
            pl.BlockSpec((_ROWS, 1), lambda i: (i, 0)),
            pl.BlockSpec((_ROWS, n), lambda i: (i, 0)),
        ],
        out_specs=pl.BlockSpec((1, 1), lambda i: (0, 0)),
        out_shape=jax.ShapeDtypeStruct((1, 1), jnp.float32),
        scratch_shapes=[pltpu.VMEM((_ROWS, n), jnp.int32)],
    )(t2, inputs)
    return out[0, 0]


# speedup vs baseline: 1.3048x; 1.0661x over previous
import functools

import jax
import jax.numpy as jnp
from jax.experimental import pallas as pl
from jax.experimental.pallas import tpu as pltpu

_DELTA = 0.2
_INT_MIN = -2147483648

_ROWS = 32
_AW = 1024


def _body(t_ref, x_ref, out_ref, *, n, num_k, inv_m):
    i = pl.program_id(0)
    tgt = t_ref[...]
    int_min = jnp.int32(_INT_MIN)
    ksel = num_k + 1
    nch = n // _AW
    tail0 = nch * _AW
    tw = n - tail0

    def from_sortable(s):
        bits = jnp.where(s >= 0, s, jnp.bitwise_not(s ^ int_min))
        return jax.lax.bitcast_convert_type(bits, jnp.float32)

    def vred(v, w, op):
        parts = [v[:, j * 128:(j + 1) * 128] for j in range(w // 128)]
        while len(parts) > 1:
            nxt = [op(a, b) for a, b in zip(parts[::2], parts[1::2])]
            if len(parts) % 2:
                nxt.append(parts[-1])
            parts = nxt
        return parts[0]

    def lsum(v):
        return jnp.sum(v, axis=1, keepdims=True)

    iota0 = jax.lax.broadcasted_iota(jnp.int32, (_ROWS, _AW), 1)
    pos_w = jnp.zeros((_ROWS, _AW), jnp.float32)
    mx_w = jnp.full((_ROWS, _AW), -jnp.inf, jnp.float32)
    for c in range(nch):
        x = x_ref[:, c * _AW:(c + 1) * _AW]
        is_t = iota0 == tgt - c * _AW
        pos_w = jnp.where(is_t, pos_w + x, pos_w)
        mx_w = jnp.maximum(mx_w, x)
    xt = x_ref[:, tail0:n]
    is_tt = jax.lax.broadcasted_iota(jnp.int32, (_ROWS, tw), 1) == tgt - tail0
    pos = (lsum(vred(pos_w, _AW, lambda a, b: a + b))
           + lsum(jnp.where(is_tt, xt, 0.0)))
    mx = jnp.maximum(
        jnp.max(vred(mx_w, _AW, jnp.maximum), axis=1, keepdims=True),
        jnp.max(xt, axis=1, keepdims=True))

    def count_ge(t_f):
        acc = jnp.zeros((_ROWS, _AW), jnp.int32)
        for c in range(nch):
            blk = x_ref[:, c * _AW:(c + 1) * _AW]
            acc = jnp.where(blk >= t_f, acc + 1, acc)
        tcnt = lsum(jnp.where(xt >= t_f, jnp.int32(1), jnp.int32(0)))
        return lsum(vred(acc, _AW, lambda a, b: a + b)) + tcnt

    def rcond(carry):
        b, _, done = carry
        return jnp.logical_and(b < 32, jnp.min(done) == 0)

    def rstep(carry):
        b, prefix, done = carry
        bit = jnp.left_shift(jnp.int32(1), 31 - b)
        cand = prefix | bit
        t_f = from_sortable(cand ^ int_min)
        run = jnp.any(jnp.logical_and(t_f <= mx, done == 0))
        cnt = jax.lax.cond(
            run, lambda: count_ge(t_f),
            lambda: jnp.zeros((_ROWS, 1), jnp.int32))
        live_take = jnp.logical_and(cnt >= ksel, done == 0)
        new_prefix = jnp.where(live_take, cand, prefix)
        new_done = jnp.where(cnt == ksel, jnp.int32(1), done)
        return b + 1, new_prefix, new_done

    _, prefix, _ = jax.lax.while_loop(
        rcond, rstep,
        (jnp.int32(0), jnp.zeros((_ROWS, 1), jnp.int32),
         jnp.zeros((_ROWS, 1), jnp.int32)))
    t_f = from_sortable(prefix ^ int_min)

    cgt_w = jnp.zeros((_ROWS, _AW), jnp.int32)
    sgt_w = jnp.zeros((_ROWS, _AW), jnp.float32)
    mn_w = jnp.full((_ROWS, _AW), jnp.inf, jnp.float32)
    for c in range(nch):
        x = x_ref[:, c * _AW:(c + 1) * _AW]
        f = (1.0 + x) ** 2
        gt = x > t_f
        cgt_w = jnp.where(gt, cgt_w + 1, cgt_w)
        sgt_w = jnp.where(gt, sgt_w + f, sgt_w)
        mn_w = jnp.where(gt, jnp.minimum(mn_w, x), mn_w)
    ft = (1.0 + xt) ** 2
    gtt = xt > t_f
    cnt_gt = (lsum(vred(cgt_w, _AW, lambda a, b: a + b))
              + lsum(jnp.where(gtt, jnp.int32(1), jnp.int32(0))))
    sum_gt = (lsum(vred(sgt_w, _AW, lambda a, b: a + b))
              + lsum(jnp.where(gtt, ft, 0.0)))
    mn_gt = jnp.minimum(
        jnp.min(vred(mn_w, _AW, jnp.minimum), axis=1, keepdims=True),
        jnp.min(jnp.where(gtt, xt, jnp.inf), axis=1, keepdims=True))

    t_true = jnp.where(cnt_gt == ksel, mn_gt, t_f)
    need = (ksel - cnt_gt).astype(jnp.float32)
    sum_topk1 = sum_gt + need * (1.0 + t_f) ** 2
    q = jnp.maximum(pos, t_true)
    top_sum = sum_topk1 - (1.0 + q) ** 2
    per_row = (1.0 - pos) ** 2 + (_DELTA / num_k) * top_sum
    blk = jnp.sum(per_row) * inv_m

    @pl.when(i == 0)
    def _():
        out_ref[...] = jnp.zeros_like(out_ref)

    out_ref[...] += blk


def kernel(inputs, targets):
    m, n = inputs.shape
    num_k = int(0.01 * (n - 1))
    t2 = targets.astype(jnp.int32).reshape(m, 1)
    body = functools.partial(_body, n=n, num_k=num_k, inv_m=1.0 / m)
    out = pl.pallas_call(
        body,
        grid=(m // _ROWS,),
        in_specs=[
            pl.BlockSpec((_ROWS, 1), lambda i: (i, 0)),
            pl.BlockSpec((_ROWS, n), lambda i: (i, 0)),
        ],
        out_specs=pl.BlockSpec((1, 1), lambda i: (0, 0)),
        out_shape=jax.ShapeDtypeStruct((1, 1), jnp.float32),
    )(t2, inputs)
    return out[0, 0]
